# baseline (speedup 1.0000x reference)
import functools

import jax
import jax.numpy as jnp
from jax import lax
from jax.experimental import pallas as pl
from jax.experimental.pallas import tpu as pltpu
from jax.experimental.pallas import tpu_sc as plsc

FEAT = 128
S = 32
T = 8
MU = 0.9
NC = 2
NS = 16
NW = NC * NS

def _sc_mesh():
    return plsc.VectorSubcoreMesh(core_axis_name="c", subcore_axis_name="s",
                                  num_cores=NC, num_subcores=NS)


def _embed_gather_body(idxn_hbm, idxd_hbm, feat_hbm, node_out, neis_out,
                       idx_v, rows_v, sem):
    c = lax.axis_index("c")
    s = lax.axis_index("s")
    w = c * NS + s
    epw = idxn_hbm.shape[2] * idxn_hbm.shape[3]
    half = epw // 2
    nsub = idxn_hbm.shape[2]
    for src_i in range(2):
        idx_hbm = (idxn_hbm, idxd_hbm)[src_i]
        out_hbm = (node_out, neis_out)[src_i]
        pltpu.sync_copy(idx_hbm.at[c, s], idx_v)
        for h in range(2):
            descs = []
            for j in range(nsub // 2):
                jj = h * (nsub // 2) + j
                descs.append(pltpu.async_copy(
                    feat_hbm.at[idx_v.at[jj]],
                    rows_v.at[pl.ds(j * 128, 128)], sem))
            for d in descs:
                d.wait()
            pltpu.sync_copy(rows_v,
                            out_hbm.at[pl.ds(w * epw + h * half, half)])


def _he_gather_body(idx_hbm, h_hbm, he_out, idx_v, rows_v, sem):
    c = lax.axis_index("c")
    s = lax.axis_index("s")
    w = c * NS + s
    nsub = idx_hbm.shape[2]
    epw = nsub * 128
    pltpu.sync_copy(idx_hbm.at[c, s], idx_v)
    descs = []
    for j in range(nsub):
        descs.append(pltpu.async_copy(
            h_hbm.at[idx_v.at[j]], rows_v.at[pl.ds(j * 128, 128)], sem))
    for d in descs:
        d.wait()
    pltpu.sync_copy(rows_v, he_out.at[pl.ds(w * epw, epw)])


def _scatter_body(out_e_hbm, loc_hbm, h_hbm, idx_v, rows_v, zero_v, accum):
    c = lax.axis_index("c")
    s = lax.axis_index("s")
    nsub = loc_hbm.shape[2]
    epw = nsub * 128
    zslab = accum.shape[0] // NS
    for r in range(zslab):
        zero_v[r, pl.ds(0, 16)] = jnp.zeros((16,), jnp.float32)
        zero_v[r, pl.ds(16, 16)] = jnp.zeros((16,), jnp.float32)
    pltpu.sync_copy(zero_v, accum.at[pl.ds(s * zslab, zslab)])
    plsc.subcore_barrier()
    pltpu.sync_copy(out_e_hbm.at[pl.ds(s * epw, epw)], rows_v)
    pltpu.sync_copy(loc_hbm.at[c, s], idx_v)
    for j in range(nsub):
        pltpu.sync_copy(rows_v.at[pl.ds(j * 128, 128)],
                        accum.at[idx_v.at[j]], add=True)
    plsc.subcore_barrier()
    vhalf = 1024
    wslab = vhalf // NS
    pltpu.sync_copy(accum.at[pl.ds(s * wslab, wslab)],
                    h_hbm.at[pl.ds(c * vhalf + s * wslab, wslab)])


def _sc_embed_gather(idxn, idxd, feat):
    E = idxn.shape[0] * idxn.shape[1] * idxn.shape[2] * idxn.shape[3]
    epw = E // NW
    return pl.kernel(
        _embed_gather_body,
        out_type=[jax.ShapeDtypeStruct((E, FEAT), jnp.float32),
                  jax.ShapeDtypeStruct((E, FEAT), jnp.float32)],
        mesh=_sc_mesh(),
        compiler_params=pltpu.CompilerParams(use_tc_tiling_on_sc=False),
        scratch_types=[pltpu.VMEM((epw // 128, 128), jnp.int32),
                       pltpu.VMEM((epw // 2, FEAT), jnp.float32),
                       pltpu.SemaphoreType.DMA],
    )(idxn, idxd, feat)


def _sc_he_gather(idx, h):
    E = idx.shape[0] * idx.shape[1] * idx.shape[2] * idx.shape[3]
    epw = E // NW
    return pl.kernel(
        _he_gather_body,
        out_type=jax.ShapeDtypeStruct((E, S), jnp.float32),
        mesh=_sc_mesh(),
        compiler_params=pltpu.CompilerParams(use_tc_tiling_on_sc=False),
        scratch_types=[pltpu.VMEM((epw // 128, 128), jnp.int32),
                       pltpu.VMEM((epw, S), jnp.float32),
                       pltpu.SemaphoreType.DMA],
    )(idx, h)


def _sc_seg_scatter(out_e, loc, V):
    E = out_e.shape[0]
    eps = E // NS
    return pl.kernel(
        _scatter_body,
        out_type=jax.ShapeDtypeStruct((V, S), jnp.float32),
        mesh=_sc_mesh(),
        compiler_params=pltpu.CompilerParams(use_tc_tiling_on_sc=False),
        scratch_types=[pltpu.VMEM((eps // 128, 128), jnp.int32),
                       pltpu.VMEM((eps, S), jnp.float32),
                       pltpu.VMEM((1088 // NS, S), jnp.float32),
                       pltpu.VMEM_SHARED((1088, S), jnp.float32)],
    )(out_e, loc)


def _agen_body(node_ref, neis_ref, et_ref, dg_ref, w1_ref, w2_ref, w3_ref,
               xib_ref, rouw_ref, roub_ref, a_ref, b_ref):
    node = node_ref[...]
    neis = neis_ref[...]
    acc = jnp.dot(node, w1_ref[...], preferred_element_type=jnp.float32)
    acc += jnp.dot(neis, w2_ref[...], preferred_element_type=jnp.float32)
    oh = (et_ref[...] == lax.broadcasted_iota(jnp.int32, (node.shape[0], T), 1)
          ).astype(jnp.float32)
    acc += jnp.dot(oh, w3_ref[...], preferred_element_type=jnp.float32)
    acc += xib_ref[...]
    scale = (MU / S) / dg_ref[...]
    a_ref[...] = jnp.tanh(acc) * scale
    b_ref[...] = jnp.tanh(
        jnp.dot(neis, rouw_ref[...], preferred_element_type=jnp.float32)
        + roub_ref[...])


def _matvec_body(a_ref, he_ref, b_ref, out_ref):
    a = a_ref[...]
    he = he_ref[...]
    out_ref[...] = jnp.sum(a * he[:, None, :], axis=2) + b_ref[...]


def _final_body(h_ref, l1w_ref, l1b_ref, out_ref):
    h = h_ref[...]
    logits = jnp.sum(h * l1w_ref[...], axis=1, keepdims=True) + l1b_ref[...]
    m = jnp.max(logits)
    e = jnp.exp(logits - m)
    attn = e / jnp.sum(e)
    out_ref[...] = jnp.tanh(jnp.sum(h * attn, axis=0, keepdims=True))


def kernel(feat_Matrix, X_Node, X_Neis, edge_type_index, dg_list,
           Xi_W, Xi_b, Rou_W, Rou_b, L1_W, L1_b):
    V = feat_Matrix.shape[0]
    E = X_Node.shape[0]
    epw = E // NW
    eps = E // NS
    vhalf = V // NC

    Xn = (X_Node - 1).astype(jnp.int32)
    Xd = (X_Neis - 1).astype(jnp.int32)
    et = (edge_type_index - 1).astype(jnp.int32)

    idxn = Xn.reshape(NC, NS, epw // 128, 128)
    idxd = Xd.reshape(NC, NS, epw // 128, 128)
    base = (jnp.arange(NC, dtype=jnp.int32) * vhalf)[:, None]
    loc = X_Neis.astype(jnp.int32)[None, :] - base
    loc = jnp.where((loc >= 0) & (loc < vhalf), loc, vhalf)
    loc = loc.reshape(NC, NS, eps // 128, 128)

    node_e, neis_e = _sc_embed_gather(idxn, idxd, feat_Matrix)

    TE2 = 256
    a2, bvec = pl.pallas_call(
        _agen_body,
        grid=(E // TE2,),
        in_specs=[
            pl.BlockSpec((TE2, FEAT), lambda i: (i, 0)),
            pl.BlockSpec((TE2, FEAT), lambda i: (i, 0)),
            pl.BlockSpec((TE2, 1), lambda i: (i, 0)),
            pl.BlockSpec((TE2, 1), lambda i: (i, 0)),
            pl.BlockSpec((FEAT, S * S), lambda i: (0, 0)),
            pl.BlockSpec((FEAT, S * S), lambda i: (0, 0)),
            pl.BlockSpec((T, S * S), lambda i: (0, 0)),
            pl.BlockSpec((1, S * S), lambda i: (0, 0)),
            pl.BlockSpec((FEAT, S), lambda i: (0, 0)),
            pl.BlockSpec((1, S), lambda i: (0, 0)),
        ],
        out_specs=[pl.BlockSpec((TE2, S * S), lambda i: (i, 0)),
                   pl.BlockSpec((TE2, S), lambda i: (i, 0))],
        out_shape=[jax.ShapeDtypeStruct((E, S * S), jnp.float32),
                   jax.ShapeDtypeStruct((E, S), jnp.float32)],
    )(node_e, neis_e, et[:, None], dg_list.astype(jnp.float32)[:, None],
      Xi_W[:, :FEAT].T, Xi_W[:, FEAT:2 * FEAT].T, Xi_W[:, 2 * FEAT:].T,
      Xi_b[None, :], Rou_W.T, Rou_b[None, :])
    a3 = a2.reshape(E, S, S)

    TE = 512
    matvec = pl.pallas_call(
        _matvec_body,
        grid=(E // TE,),
        in_specs=[
            pl.BlockSpec((TE, S, S), lambda i: (i, 0, 0)),
            pl.BlockSpec((TE, S), lambda i: (i, 0)),
            pl.BlockSpec((TE, S), lambda i: (i, 0)),
        ],
        out_specs=pl.BlockSpec((TE, S), lambda i: (i, 0)),
        out_shape=jax.ShapeDtypeStruct((E, S), jnp.float32),
    )

    h = jax.random.uniform(jax.random.key(1), (V, S), dtype=jnp.float32)
    he = _sc_he_gather(idxn, h)
    for t in range(T):
        out_e = matvec(a3, he, bvec)
        h = _sc_seg_scatter(out_e, loc, V)
        if t < T - 1:
            he = _sc_he_gather(idxn, h)

    graph = pl.pallas_call(
        _final_body,
        in_specs=[pl.BlockSpec((V, S), lambda: (0, 0)),
                  pl.BlockSpec((1, S), lambda: (0, 0)),
                  pl.BlockSpec((1, 1), lambda: (0, 0))],
        out_specs=pl.BlockSpec((1, S), lambda: (0, 0)),
        out_shape=jax.ShapeDtypeStruct((1, S), jnp.float32),
    )(h, L1_W, L1_b[None, :])
    return graph.reshape(S)


# speedup vs baseline: 3.2684x; 3.2684x over previous
import functools

import jax
import jax.numpy as jnp
from jax import lax
from jax.experimental import pallas as pl
from jax.experimental.pallas import tpu as pltpu
from jax.experimental.pallas import tpu_sc as plsc

FEAT = 128
S = 32
T = 8
MU = 0.9
NC = 2
NS = 16
NW = NC * NS

def _sc_mesh():
    return plsc.VectorSubcoreMesh(core_axis_name="c", subcore_axis_name="s",
                                  num_cores=NC, num_subcores=NS)


def _embed_gather_body(idxn_hbm, idxd_hbm, feat_hbm, node_out, neis_out,
                       idx_v, rows_v, sem):
    c = lax.axis_index("c")
    s = lax.axis_index("s")
    w = c * NS + s
    epw = idxn_hbm.shape[2] * idxn_hbm.shape[3]
    half = epw // 2
    nsub = idxn_hbm.shape[2]
    for src_i in range(2):
        idx_hbm = (idxn_hbm, idxd_hbm)[src_i]
        out_hbm = (node_out, neis_out)[src_i]
        pltpu.sync_copy(idx_hbm.at[c, s], idx_v)
        for h in range(2):
            descs = []
            for j in range(nsub // 2):
                jj = h * (nsub // 2) + j
                descs.append(pltpu.async_copy(
                    feat_hbm.at[idx_v.at[jj]],
                    rows_v.at[pl.ds(j * 128, 128)], sem))
            for d in descs:
                d.wait()
            pltpu.sync_copy(rows_v,
                            out_hbm.at[pl.ds(w * epw + h * half, half)])


def _he_gather_body(idx_hbm, h_hbm, he_out, idx_v, rows_v, sem):
    c = lax.axis_index("c")
    s = lax.axis_index("s")
    w = c * NS + s
    nsub = idx_hbm.shape[2]
    epw = nsub * 128
    pltpu.sync_copy(idx_hbm.at[c, s], idx_v)
    descs = []
    for j in range(nsub):
        descs.append(pltpu.async_copy(
            h_hbm.at[idx_v.at[j]], rows_v.at[pl.ds(j * 128, 128)], sem))
    for d in descs:
        d.wait()
    pltpu.sync_copy(rows_v, he_out.at[pl.ds(w * epw, epw)])


def _scatter_body(out_e_hbm, loc_hbm, h_hbm, idx_v, rows_v, zero_v, accum):
    c = lax.axis_index("c")
    s = lax.axis_index("s")
    nsub = loc_hbm.shape[2]
    epw = nsub * 128
    zslab = accum.shape[0] // NS
    for r in range(zslab):
        zero_v[r, pl.ds(0, 16)] = jnp.zeros((16,), jnp.float32)
        zero_v[r, pl.ds(16, 16)] = jnp.zeros((16,), jnp.float32)
    pltpu.sync_copy(zero_v, accum.at[pl.ds(s * zslab, zslab)])
    plsc.subcore_barrier()
    pltpu.sync_copy(out_e_hbm.at[pl.ds(s * epw, epw)], rows_v)
    pltpu.sync_copy(loc_hbm.at[c, s], idx_v)
    for j in range(nsub):
        pltpu.sync_copy(rows_v.at[pl.ds(j * 128, 128)],
                        accum.at[idx_v.at[j]], add=True)
    plsc.subcore_barrier()
    vhalf = 1024
    wslab = vhalf // NS
    pltpu.sync_copy(accum.at[pl.ds(s * wslab, wslab)],
                    h_hbm.at[pl.ds(c * vhalf + s * wslab, wslab)])


def _sc_embed_gather(idxn, idxd, feat):
    E = idxn.shape[0] * idxn.shape[1] * idxn.shape[2] * idxn.shape[3]
    epw = E // NW
    return pl.kernel(
        _embed_gather_body,
        out_type=[jax.ShapeDtypeStruct((E, FEAT), jnp.float32),
                  jax.ShapeDtypeStruct((E, FEAT), jnp.float32)],
        mesh=_sc_mesh(),
        compiler_params=pltpu.CompilerParams(use_tc_tiling_on_sc=False),
        scratch_types=[pltpu.VMEM((epw // 128, 128), jnp.int32),
                       pltpu.VMEM((epw // 2, FEAT), jnp.float32),
                       pltpu.SemaphoreType.DMA],
    )(idxn, idxd, feat)


def _sc_he_gather(idx, h):
    E = idx.shape[0] * idx.shape[1] * idx.shape[2] * idx.shape[3]
    epw = E // NW
    return pl.kernel(
        _he_gather_body,
        out_type=jax.ShapeDtypeStruct((E, S), jnp.float32),
        mesh=_sc_mesh(),
        compiler_params=pltpu.CompilerParams(use_tc_tiling_on_sc=False),
        scratch_types=[pltpu.VMEM((epw // 128, 128), jnp.int32),
                       pltpu.VMEM((epw, S), jnp.float32),
                       pltpu.SemaphoreType.DMA],
    )(idx, h)


def _sc_seg_scatter(out_e, loc, V):
    E = out_e.shape[0]
    eps = E // NS
    return pl.kernel(
        _scatter_body,
        out_type=jax.ShapeDtypeStruct((V, S), jnp.float32),
        mesh=_sc_mesh(),
        compiler_params=pltpu.CompilerParams(use_tc_tiling_on_sc=False),
        scratch_types=[pltpu.VMEM((eps // 128, 128), jnp.int32),
                       pltpu.VMEM((eps, S), jnp.float32),
                       pltpu.VMEM((1088 // NS, S), jnp.float32),
                       pltpu.VMEM_SHARED((1088, S), jnp.float32)],
    )(out_e, loc)


def _agen_body(node_ref, neis_ref, et_ref, dg_ref, w1_ref, w2_ref, w3_ref,
               xib_ref, rouw_ref, roub_ref, a_ref, b_ref):
    node = node_ref[...]
    neis = neis_ref[...]
    acc = jnp.dot(node, w1_ref[...], preferred_element_type=jnp.float32)
    acc += jnp.dot(neis, w2_ref[...], preferred_element_type=jnp.float32)
    oh = (et_ref[...] == lax.broadcasted_iota(jnp.int32, (node.shape[0], T), 1)
          ).astype(jnp.float32)
    acc += jnp.dot(oh, w3_ref[...], preferred_element_type=jnp.float32)
    acc += xib_ref[...]
    scale = (MU / S) / dg_ref[...]
    a_ref[...] = jnp.tanh(acc) * scale
    b_ref[...] = jnp.tanh(
        jnp.dot(neis, rouw_ref[...], preferred_element_type=jnp.float32)
        + roub_ref[...])


def _matvec_body(a_ref, he_ref, b_ref, rep_ref, fold_ref, out_ref):
    he_rep = jnp.dot(he_ref[...], rep_ref[...],
                     preferred_element_type=jnp.float32)
    out_ref[...] = jnp.dot(a_ref[...] * he_rep, fold_ref[...],
                           preferred_element_type=jnp.float32) + b_ref[...]


def _final_body(h_ref, l1w_ref, l1b_ref, out_ref):
    h = h_ref[...]
    logits = jnp.sum(h * l1w_ref[...], axis=1, keepdims=True) + l1b_ref[...]
    m = jnp.max(logits)
    e = jnp.exp(logits - m)
    attn = e / jnp.sum(e)
    out_ref[...] = jnp.tanh(jnp.sum(h * attn, axis=0, keepdims=True))


def kernel(feat_Matrix, X_Node, X_Neis, edge_type_index, dg_list,
           Xi_W, Xi_b, Rou_W, Rou_b, L1_W, L1_b):
    V = feat_Matrix.shape[0]
    E = X_Node.shape[0]
    epw = E // NW
    eps = E // NS
    vhalf = V // NC

    Xn = (X_Node - 1).astype(jnp.int32)
    Xd = (X_Neis - 1).astype(jnp.int32)
    et = (edge_type_index - 1).astype(jnp.int32)

    idxn = Xn.reshape(NC, NS, epw // 128, 128)
    idxd = Xd.reshape(NC, NS, epw // 128, 128)
    base = (jnp.arange(NC, dtype=jnp.int32) * vhalf)[:, None]
    loc = X_Neis.astype(jnp.int32)[None, :] - base
    loc = jnp.where((loc >= 0) & (loc < vhalf), loc, vhalf)
    loc = loc.reshape(NC, NS, eps // 128, 128)

    node_e, neis_e = _sc_embed_gather(idxn, idxd, feat_Matrix)

    TE2 = 256
    a2, bvec = pl.pallas_call(
        _agen_body,
        grid=(E // TE2,),
        in_specs=[
            pl.BlockSpec((TE2, FEAT), lambda i: (i, 0)),
            pl.BlockSpec((TE2, FEAT), lambda i: (i, 0)),
            pl.BlockSpec((TE2, 1), lambda i: (i, 0)),
            pl.BlockSpec((TE2, 1), lambda i: (i, 0)),
            pl.BlockSpec((FEAT, S * S), lambda i: (0, 0)),
            pl.BlockSpec((FEAT, S * S), lambda i: (0, 0)),
            pl.BlockSpec((T, S * S), lambda i: (0, 0)),
            pl.BlockSpec((1, S * S), lambda i: (0, 0)),
            pl.BlockSpec((FEAT, S), lambda i: (0, 0)),
            pl.BlockSpec((1, S), lambda i: (0, 0)),
        ],
        out_specs=[pl.BlockSpec((TE2, S * S), lambda i: (i, 0)),
                   pl.BlockSpec((TE2, S), lambda i: (i, 0))],
        out_shape=[jax.ShapeDtypeStruct((E, S * S), jnp.float32),
                   jax.ShapeDtypeStruct((E, S), jnp.float32)],
    )(node_e, neis_e, et[:, None], dg_list.astype(jnp.float32)[:, None],
      Xi_W[:, :FEAT].T, Xi_W[:, FEAT:2 * FEAT].T, Xi_W[:, 2 * FEAT:].T,
      Xi_b[None, :], Rou_W.T, Rou_b[None, :])
    rep_c = jnp.tile(jnp.eye(S, dtype=jnp.float32), (1, S))
    fold_c = jnp.repeat(jnp.eye(S, dtype=jnp.float32), S, axis=0)

    TE = 512
    matvec = pl.pallas_call(
        _matvec_body,
        grid=(E // TE,),
        in_specs=[
            pl.BlockSpec((TE, S * S), lambda i: (i, 0)),
            pl.BlockSpec((TE, S), lambda i: (i, 0)),
            pl.BlockSpec((TE, S), lambda i: (i, 0)),
            pl.BlockSpec((S, S * S), lambda i: (0, 0)),
            pl.BlockSpec((S * S, S), lambda i: (0, 0)),
        ],
        out_specs=pl.BlockSpec((TE, S), lambda i: (i, 0)),
        out_shape=jax.ShapeDtypeStruct((E, S), jnp.float32),
    )

    h = jax.random.uniform(jax.random.key(1), (V, S), dtype=jnp.float32)
    he = _sc_he_gather(idxn, h)
    for t in range(T):
        out_e = matvec(a2, he, bvec, rep_c, fold_c)
        h = _sc_seg_scatter(out_e, loc, V)
        if t < T - 1:
            he = _sc_he_gather(idxn, h)

    graph = pl.pallas_call(
        _final_body,
        in_specs=[pl.BlockSpec((V, S), lambda: (0, 0)),
                  pl.BlockSpec((1, S), lambda: (0, 0)),
                  pl.BlockSpec((1, 1), lambda: (0, 0))],
        out_specs=pl.BlockSpec((1, S), lambda: (0, 0)),
        out_shape=jax.ShapeDtypeStruct((1, S), jnp.float32),
    )(h, L1_W, L1_b[None, :])
    return graph.reshape(S)


# speedup vs baseline: 3.5409x; 1.0834x over previous
import functools

import jax
import jax.numpy as jnp
from jax import lax
from jax.experimental import pallas as pl
from jax.experimental.pallas import tpu as pltpu
from jax.experimental.pallas import tpu_sc as plsc

FEAT = 128
S = 32
T = 8
MU = 0.9
NC = 2
NS = 16
NW = NC * NS

def _sc_mesh():
    return plsc.VectorSubcoreMesh(core_axis_name="c", subcore_axis_name="s",
                                  num_cores=NC, num_subcores=NS)


def _embed_gather_body(idxn_hbm, idxd_hbm, feat_hbm, node_out, neis_out,
                       idx_v, rows_v, sem):
    c = lax.axis_index("c")
    s = lax.axis_index("s")
    w = c * NS + s
    epw = idxn_hbm.shape[2] * idxn_hbm.shape[3]
    half = epw // 2
    nsub = idxn_hbm.shape[2]
    for src_i in range(2):
        idx_hbm = (idxn_hbm, idxd_hbm)[src_i]
        out_hbm = (node_out, neis_out)[src_i]
        pltpu.sync_copy(idx_hbm.at[c, s], idx_v)
        for h in range(2):
            descs = []
            for j in range(nsub // 2):
                jj = h * (nsub // 2) + j
                descs.append(pltpu.async_copy(
                    feat_hbm.at[idx_v.at[jj]],
                    rows_v.at[pl.ds(j * 128, 128)], sem))
            for d in descs:
                d.wait()
            pltpu.sync_copy(rows_v,
                            out_hbm.at[pl.ds(w * epw + h * half, half)])


def _he_gather_body(idx_hbm, h_hbm, he_out, idx_v, rows_v, sem):
    c = lax.axis_index("c")
    s = lax.axis_index("s")
    w = c * NS + s
    nsub = idx_hbm.shape[2]
    epw = nsub * 128
    pltpu.sync_copy(idx_hbm.at[c, s], idx_v)
    descs = []
    for j in range(nsub):
        descs.append(pltpu.async_copy(
            h_hbm.at[idx_v.at[j]], rows_v.at[pl.ds(j * 128, 128)], sem))
    for d in descs:
        d.wait()
    pltpu.sync_copy(rows_v, he_out.at[pl.ds(w * epw, epw)])


def _scatter_body(out_e_hbm, loc_hbm, h_hbm, idx_v, rows_v, zero_v, accum):
    c = lax.axis_index("c")
    s = lax.axis_index("s")
    nsub = loc_hbm.shape[2]
    epw = nsub * 128
    zslab = accum.shape[0] // NS
    for r in range(zslab):
        zero_v[r, pl.ds(0, 16)] = jnp.zeros((16,), jnp.float32)
        zero_v[r, pl.ds(16, 16)] = jnp.zeros((16,), jnp.float32)
    pltpu.sync_copy(zero_v, accum.at[pl.ds(s * zslab, zslab)])
    plsc.subcore_barrier()
    pltpu.sync_copy(out_e_hbm.at[pl.ds(s * epw, epw)], rows_v)
    pltpu.sync_copy(loc_hbm.at[c, s], idx_v)
    for j in range(nsub):
        pltpu.sync_copy(rows_v.at[pl.ds(j * 128, 128)],
                        accum.at[idx_v.at[j]], add=True)
    plsc.subcore_barrier()
    vhalf = 1024
    wslab = vhalf // NS
    pltpu.sync_copy(accum.at[pl.ds(s * wslab, wslab)],
                    h_hbm.at[pl.ds(c * vhalf + s * wslab, wslab)])


def _sc_embed_gather(idxn, idxd, feat):
    E = idxn.shape[0] * idxn.shape[1] * idxn.shape[2] * idxn.shape[3]
    epw = E // NW
    return pl.kernel(
        _embed_gather_body,
        out_type=[jax.ShapeDtypeStruct((E, FEAT), jnp.float32),
                  jax.ShapeDtypeStruct((E, FEAT), jnp.float32)],
        mesh=_sc_mesh(),
        compiler_params=pltpu.CompilerParams(use_tc_tiling_on_sc=False),
        scratch_types=[pltpu.VMEM((epw // 128, 128), jnp.int32),
                       pltpu.VMEM((epw // 2, FEAT), jnp.float32),
                       pltpu.SemaphoreType.DMA],
    )(idxn, idxd, feat)


def _sc_he_gather(idx, h):
    E = idx.shape[0] * idx.shape[1] * idx.shape[2] * idx.shape[3]
    epw = E // NW
    return pl.kernel(
        _he_gather_body,
        out_type=jax.ShapeDtypeStruct((E, S), jnp.float32),
        mesh=_sc_mesh(),
        compiler_params=pltpu.CompilerParams(use_tc_tiling_on_sc=False),
        scratch_types=[pltpu.VMEM((epw // 128, 128), jnp.int32),
                       pltpu.VMEM((epw, S), jnp.float32),
                       pltpu.SemaphoreType.DMA],
    )(idx, h)


def _sc_seg_scatter(out_e, loc, V):
    E = out_e.shape[0]
    eps = E // NS
    return pl.kernel(
        _scatter_body,
        out_type=jax.ShapeDtypeStruct((V, S), jnp.float32),
        mesh=_sc_mesh(),
        compiler_params=pltpu.CompilerParams(use_tc_tiling_on_sc=False),
        scratch_types=[pltpu.VMEM((eps // 128, 128), jnp.int32),
                       pltpu.VMEM((eps, S), jnp.float32),
                       pltpu.VMEM((1088 // NS, S), jnp.float32),
                       pltpu.VMEM_SHARED((1088, S), jnp.float32)],
    )(out_e, loc)


def _agen_body(node_ref, neis_ref, et_ref, dg_ref, w1_ref, w2_ref, w3_ref,
               xib_ref, rouw_ref, roub_ref, a_ref, b_ref):
    node = node_ref[...]
    neis = neis_ref[...]
    acc = jnp.dot(node, w1_ref[...], preferred_element_type=jnp.float32)
    acc += jnp.dot(neis, w2_ref[...], preferred_element_type=jnp.float32)
    oh = (et_ref[...] == lax.broadcasted_iota(jnp.int32, (node.shape[0], T), 1)
          ).astype(jnp.float32)
    acc += jnp.dot(oh, w3_ref[...], preferred_element_type=jnp.float32)
    acc += xib_ref[...]
    scale = (MU / S) / dg_ref[...]
    a_ref[...] = (jnp.tanh(acc) * scale).astype(jnp.bfloat16)
    b_ref[...] = jnp.tanh(
        jnp.dot(neis, rouw_ref[...], preferred_element_type=jnp.float32)
        + roub_ref[...])


def _matvec_body(a_ref, he_ref, b_ref, rep_ref, fold_ref, out_ref):
    he_rep = jnp.dot(he_ref[...], rep_ref[...],
                     preferred_element_type=jnp.float32)
    prod = a_ref[...].astype(jnp.float32) * he_rep
    out_ref[...] = jnp.dot(prod, fold_ref[...],
                           preferred_element_type=jnp.float32) + b_ref[...]


def _final_body(h_ref, l1w_ref, l1b_ref, out_ref):
    h = h_ref[...]
    logits = jnp.sum(h * l1w_ref[...], axis=1, keepdims=True) + l1b_ref[...]
    m = jnp.max(logits)
    e = jnp.exp(logits - m)
    attn = e / jnp.sum(e)
    out_ref[...] = jnp.tanh(jnp.sum(h * attn, axis=0, keepdims=True))


def kernel(feat_Matrix, X_Node, X_Neis, edge_type_index, dg_list,
           Xi_W, Xi_b, Rou_W, Rou_b, L1_W, L1_b):
    V = feat_Matrix.shape[0]
    E = X_Node.shape[0]
    epw = E // NW
    eps = E // NS
    vhalf = V // NC

    Xn = (X_Node - 1).astype(jnp.int32)
    Xd = (X_Neis - 1).astype(jnp.int32)
    et = (edge_type_index - 1).astype(jnp.int32)

    idxn = Xn.reshape(NC, NS, epw // 128, 128)
    idxd = Xd.reshape(NC, NS, epw // 128, 128)
    base = (jnp.arange(NC, dtype=jnp.int32) * vhalf)[:, None]
    loc = X_Neis.astype(jnp.int32)[None, :] - base
    loc = jnp.where((loc >= 0) & (loc < vhalf), loc, vhalf)
    loc = loc.reshape(NC, NS, eps // 128, 128)

    node_e, neis_e = _sc_embed_gather(idxn, idxd, feat_Matrix)

    TE2 = 256
    a2, bvec = pl.pallas_call(
        _agen_body,
        grid=(E // TE2,),
        in_specs=[
            pl.BlockSpec((TE2, FEAT), lambda i: (i, 0)),
            pl.BlockSpec((TE2, FEAT), lambda i: (i, 0)),
            pl.BlockSpec((TE2, 1), lambda i: (i, 0)),
            pl.BlockSpec((TE2, 1), lambda i: (i, 0)),
            pl.BlockSpec((FEAT, S * S), lambda i: (0, 0)),
            pl.BlockSpec((FEAT, S * S), lambda i: (0, 0)),
            pl.BlockSpec((T, S * S), lambda i: (0, 0)),
            pl.BlockSpec((1, S * S), lambda i: (0, 0)),
            pl.BlockSpec((FEAT, S), lambda i: (0, 0)),
            pl.BlockSpec((1, S), lambda i: (0, 0)),
        ],
        out_specs=[pl.BlockSpec((TE2, S * S), lambda i: (i, 0)),
                   pl.BlockSpec((TE2, S), lambda i: (i, 0))],
        out_shape=[jax.ShapeDtypeStruct((E, S * S), jnp.bfloat16),
                   jax.ShapeDtypeStruct((E, S), jnp.float32)],
    )(node_e, neis_e, et[:, None], dg_list.astype(jnp.float32)[:, None],
      Xi_W[:, :FEAT].T, Xi_W[:, FEAT:2 * FEAT].T, Xi_W[:, 2 * FEAT:].T,
      Xi_b[None, :], Rou_W.T, Rou_b[None, :])
    rep_c = jnp.tile(jnp.eye(S, dtype=jnp.float32), (1, S))
    fold_c = jnp.repeat(jnp.eye(S, dtype=jnp.float32), S, axis=0)

    TE = 512
    matvec = pl.pallas_call(
        _matvec_body,
        grid=(E // TE,),
        in_specs=[
            pl.BlockSpec((TE, S * S), lambda i: (i, 0)),
            pl.BlockSpec((TE, S), lambda i: (i, 0)),
            pl.BlockSpec((TE, S), lambda i: (i, 0)),
            pl.BlockSpec((S, S * S), lambda i: (0, 0)),
            pl.BlockSpec((S * S, S), lambda i: (0, 0)),
        ],
        out_specs=pl.BlockSpec((TE, S), lambda i: (i, 0)),
        out_shape=jax.ShapeDtypeStruct((E, S), jnp.float32),
    )

    h = jax.random.uniform(jax.random.key(1), (V, S), dtype=jnp.float32)
    he = _sc_he_gather(idxn, h)
    for t in range(T):
        out_e = matvec(a2, he, bvec, rep_c, fold_c)
        h = _sc_seg_scatter(out_e, loc, V)
        if t < T - 1:
            he = _sc_he_gather(idxn, h)

    graph = pl.pallas_call(
        _final_body,
        in_specs=[pl.BlockSpec((V, S), lambda: (0, 0)),
                  pl.BlockSpec((1, S), lambda: (0, 0)),
                  pl.BlockSpec((1, 1), lambda: (0, 0))],
        out_specs=pl.BlockSpec((1, S), lambda: (0, 0)),
        out_shape=jax.ShapeDtypeStruct((1, S), jnp.float32),
    )(h, L1_W, L1_b[None, :])
    return graph.reshape(S)


# speedup vs baseline: 3.9961x; 1.1285x over previous
import functools

import jax
import jax.numpy as jnp
from jax import lax
from jax.experimental import pallas as pl
from jax.experimental.pallas import tpu as pltpu
from jax.experimental.pallas import tpu_sc as plsc

FEAT = 128
S = 32
T = 8
MU = 0.9
NC = 2
NS = 16
NW = NC * NS

def _sc_mesh():
    return plsc.VectorSubcoreMesh(core_axis_name="c", subcore_axis_name="s",
                                  num_cores=NC, num_subcores=NS)


def _embed_gather_body(idxn_hbm, idxd_hbm, feat_hbm, node_out, neis_out,
                       idx_v, rows_v, sem):
    c = lax.axis_index("c")
    s = lax.axis_index("s")
    w = c * NS + s
    epw = idxn_hbm.shape[2] * idxn_hbm.shape[3]
    half = epw // 2
    nsub = idxn_hbm.shape[2]
    for src_i in range(2):
        idx_hbm = (idxn_hbm, idxd_hbm)[src_i]
        out_hbm = (node_out, neis_out)[src_i]
        pltpu.sync_copy(idx_hbm.at[c, s], idx_v)
        for h in range(2):
            descs = []
            for j in range(nsub // 2):
                jj = h * (nsub // 2) + j
                descs.append(pltpu.async_copy(
                    feat_hbm.at[idx_v.at[jj]],
                    rows_v.at[pl.ds(j * 128, 128)], sem))
            for d in descs:
                d.wait()
            pltpu.sync_copy(rows_v,
                            out_hbm.at[pl.ds(w * epw + h * half, half)])


def _he_gather_body(idx_hbm, h_hbm, he_out, idx_v, rows_v, sem):
    c = lax.axis_index("c")
    s = lax.axis_index("s")
    w = c * NS + s
    nsub = idx_hbm.shape[2]
    epw = nsub * 128
    pltpu.sync_copy(idx_hbm.at[c, s], idx_v)
    descs = []
    for j in range(nsub):
        descs.append(pltpu.async_copy(
            h_hbm.at[idx_v.at[j]], rows_v.at[pl.ds(j * 128, 128)], sem))
    for d in descs:
        d.wait()
    pltpu.sync_copy(rows_v, he_out.at[pl.ds(w * epw, epw)])


def _aggregate_body(out_e_hbm, loc_hbm, gidx_hbm, h_hbm, he_out,
                    idx_v, gidx_v, rows_v, grow_v, zero_v, accum, sem):
    c = lax.axis_index("c")
    s = lax.axis_index("s")
    nsub = loc_hbm.shape[1]
    eps = nsub * 128
    zslab = accum.shape[0] // NS
    for r in range(zslab):
        zero_v[r, pl.ds(0, 16)] = jnp.zeros((16,), jnp.float32)
        zero_v[r, pl.ds(16, 16)] = jnp.zeros((16,), jnp.float32)
    pltpu.sync_copy(zero_v, accum.at[pl.ds(s * zslab, zslab)])
    plsc.subcore_barrier()
    pltpu.sync_copy(out_e_hbm.at[pl.ds(s * eps, eps)], rows_v)
    pltpu.sync_copy(loc_hbm.at[s], idx_v)
    for j in range(nsub):
        pltpu.sync_copy(rows_v.at[pl.ds(j * 128, 128)],
                        accum.at[idx_v.at[j]], add=True)
    plsc.subcore_barrier()
    vhalf = 1024
    wslab = vhalf // NS
    pltpu.sync_copy(accum.at[pl.ds(c * vhalf + s * wslab, wslab)],
                    h_hbm.at[pl.ds(c * vhalf + s * wslab, wslab)])
    w = c * NS + s
    gsub = gidx_hbm.shape[2]
    epw = gsub * 128
    pltpu.sync_copy(gidx_hbm.at[c, s], gidx_v)
    descs = []
    for j in range(gsub):
        descs.append(pltpu.async_copy(
            accum.at[gidx_v.at[j]], grow_v.at[pl.ds(j * 128, 128)], sem))
    for d in descs:
        d.wait()
    pltpu.sync_copy(grow_v, he_out.at[pl.ds(w * epw, epw)])


def _sc_aggregate(out_e, loc3, gidx, V):
    E = out_e.shape[0]
    eps = E // NS
    epw = E // NW
    return pl.kernel(
        _aggregate_body,
        out_type=[jax.ShapeDtypeStruct((V, S), jnp.float32),
                  jax.ShapeDtypeStruct((E, S), jnp.float32)],
        mesh=_sc_mesh(),
        compiler_params=pltpu.CompilerParams(use_tc_tiling_on_sc=False),
        scratch_types=[pltpu.VMEM((eps // 128, 128), jnp.int32),
                       pltpu.VMEM((epw // 128, 128), jnp.int32),
                       pltpu.VMEM((eps, S), jnp.float32),
                       pltpu.VMEM((epw, S), jnp.float32),
                       pltpu.VMEM((2080 // NS, S), jnp.float32),
                       pltpu.VMEM_SHARED((2080, S), jnp.float32),
                       pltpu.SemaphoreType.DMA],
    )(out_e, loc3, gidx)


def _sc_embed_gather(idxn, idxd, feat):
    E = idxn.shape[0] * idxn.shape[1] * idxn.shape[2] * idxn.shape[3]
    epw = E // NW
    return pl.kernel(
        _embed_gather_body,
        out_type=[jax.ShapeDtypeStruct((E, FEAT), jnp.float32),
                  jax.ShapeDtypeStruct((E, FEAT), jnp.float32)],
        mesh=_sc_mesh(),
        compiler_params=pltpu.CompilerParams(use_tc_tiling_on_sc=False),
        scratch_types=[pltpu.VMEM((epw // 128, 128), jnp.int32),
                       pltpu.VMEM((epw // 2, FEAT), jnp.float32),
                       pltpu.SemaphoreType.DMA],
    )(idxn, idxd, feat)


def _sc_he_gather(idx, h):
    E = idx.shape[0] * idx.shape[1] * idx.shape[2] * idx.shape[3]
    epw = E // NW
    return pl.kernel(
        _he_gather_body,
        out_type=jax.ShapeDtypeStruct((E, S), jnp.float32),
        mesh=_sc_mesh(),
        compiler_params=pltpu.CompilerParams(use_tc_tiling_on_sc=False),
        scratch_types=[pltpu.VMEM((epw // 128, 128), jnp.int32),
                       pltpu.VMEM((epw, S), jnp.float32),
                       pltpu.SemaphoreType.DMA],
    )(idx, h)


def _agen_body(node_ref, neis_ref, et_ref, dg_ref, w1_ref, w2_ref, w3_ref,
               xib_ref, rouw_ref, roub_ref, a_ref, b_ref):
    node = node_ref[...]
    neis = neis_ref[...]
    acc = jnp.dot(node, w1_ref[...], preferred_element_type=jnp.float32)
    acc += jnp.dot(neis, w2_ref[...], preferred_element_type=jnp.float32)
    oh = (et_ref[...] == lax.broadcasted_iota(jnp.int32, (node.shape[0], T), 1)
          ).astype(jnp.float32)
    acc += jnp.dot(oh, w3_ref[...], preferred_element_type=jnp.float32)
    acc += xib_ref[...]
    scale = (MU / S) / dg_ref[...]
    a_ref[...] = (jnp.tanh(acc) * scale).astype(jnp.bfloat16)
    b_ref[...] = jnp.tanh(
        jnp.dot(neis, rouw_ref[...], preferred_element_type=jnp.float32)
        + roub_ref[...])


def _matvec_body(a_ref, he_ref, b_ref, rep_ref, fold_ref, out_ref):
    he_rep = jnp.dot(he_ref[...], rep_ref[...],
                     preferred_element_type=jnp.float32)
    prod = a_ref[...].astype(jnp.float32) * he_rep
    out_ref[...] = jnp.dot(prod, fold_ref[...],
                           preferred_element_type=jnp.float32) + b_ref[...]


def _final_body(h_ref, l1w_ref, l1b_ref, out_ref):
    h = h_ref[...]
    logits = jnp.sum(h * l1w_ref[...], axis=1, keepdims=True) + l1b_ref[...]
    m = jnp.max(logits)
    e = jnp.exp(logits - m)
    attn = e / jnp.sum(e)
    out_ref[...] = jnp.tanh(jnp.sum(h * attn, axis=0, keepdims=True))


def kernel(feat_Matrix, X_Node, X_Neis, edge_type_index, dg_list,
           Xi_W, Xi_b, Rou_W, Rou_b, L1_W, L1_b):
    V = feat_Matrix.shape[0]
    E = X_Node.shape[0]
    epw = E // NW
    eps = E // NS

    Xn = (X_Node - 1).astype(jnp.int32)
    Xd = (X_Neis - 1).astype(jnp.int32)
    et = (edge_type_index - 1).astype(jnp.int32)

    idxn = Xn.reshape(NC, NS, epw // 128, 128)
    idxd = Xd.reshape(NC, NS, epw // 128, 128)
    loc3 = X_Neis.astype(jnp.int32).reshape(NS, eps // 128, 128)

    node_e, neis_e = _sc_embed_gather(idxn, idxd, feat_Matrix)

    TE2 = 256
    a2, bvec = pl.pallas_call(
        _agen_body,
        grid=(E // TE2,),
        in_specs=[
            pl.BlockSpec((TE2, FEAT), lambda i: (i, 0)),
            pl.BlockSpec((TE2, FEAT), lambda i: (i, 0)),
            pl.BlockSpec((TE2, 1), lambda i: (i, 0)),
            pl.BlockSpec((TE2, 1), lambda i: (i, 0)),
            pl.BlockSpec((FEAT, S * S), lambda i: (0, 0)),
            pl.BlockSpec((FEAT, S * S), lambda i: (0, 0)),
            pl.BlockSpec((T, S * S), lambda i: (0, 0)),
            pl.BlockSpec((1, S * S), lambda i: (0, 0)),
            pl.BlockSpec((FEAT, S), lambda i: (0, 0)),
            pl.BlockSpec((1, S), lambda i: (0, 0)),
        ],
        out_specs=[pl.BlockSpec((TE2, S * S), lambda i: (i, 0)),
                   pl.BlockSpec((TE2, S), lambda i: (i, 0))],
        out_shape=[jax.ShapeDtypeStruct((E, S * S), jnp.bfloat16),
                   jax.ShapeDtypeStruct((E, S), jnp.float32)],
    )(node_e, neis_e, et[:, None], dg_list.astype(jnp.float32)[:, None],
      Xi_W[:, :FEAT].T, Xi_W[:, FEAT:2 * FEAT].T, Xi_W[:, 2 * FEAT:].T,
      Xi_b[None, :], Rou_W.T, Rou_b[None, :])
    rep_c = jnp.tile(jnp.eye(S, dtype=jnp.float32), (1, S))
    fold_c = jnp.repeat(jnp.eye(S, dtype=jnp.float32), S, axis=0)

    TE = 512
    matvec = pl.pallas_call(
        _matvec_body,
        grid=(E // TE,),
        in_specs=[
            pl.BlockSpec((TE, S * S), lambda i: (i, 0)),
            pl.BlockSpec((TE, S), lambda i: (i, 0)),
            pl.BlockSpec((TE, S), lambda i: (i, 0)),
            pl.BlockSpec((S, S * S), lambda i: (0, 0)),
            pl.BlockSpec((S * S, S), lambda i: (0, 0)),
        ],
        out_specs=pl.BlockSpec((TE, S), lambda i: (i, 0)),
        out_shape=jax.ShapeDtypeStruct((E, S), jnp.float32),
    )

    h = jax.random.uniform(jax.random.key(1), (V, S), dtype=jnp.float32)
    he = _sc_he_gather(idxn, h)
    for _ in range(T):
        out_e = matvec(a2, he, bvec, rep_c, fold_c)
        h, he = _sc_aggregate(out_e, loc3, idxn, V)

    graph = pl.pallas_call(
        _final_body,
        in_specs=[pl.BlockSpec((V, S), lambda: (0, 0)),
                  pl.BlockSpec((1, S), lambda: (0, 0)),
                  pl.BlockSpec((1, 1), lambda: (0, 0))],
        out_specs=pl.BlockSpec((1, S), lambda: (0, 0)),
        out_shape=jax.ShapeDtypeStruct((1, S), jnp.float32),
    )(h, L1_W, L1_b[None, :])
    return graph.reshape(S)


# speedup vs baseline: 4.6364x; 1.1602x over previous
import functools

import jax
import jax.numpy as jnp
from jax import lax
from jax.experimental import pallas as pl
from jax.experimental.pallas import tpu as pltpu
from jax.experimental.pallas import tpu_sc as plsc

FEAT = 128
S = 32
T = 8
MU = 0.9
NC = 2
NS = 16
NW = NC * NS

def _sc_mesh():
    return plsc.VectorSubcoreMesh(core_axis_name="c", subcore_axis_name="s",
                                  num_cores=NC, num_subcores=NS)


def _embed_gather_body(idxn_hbm, idxd_hbm, feat_hbm, node_out, neis_out,
                       idx_v, rows_v, sem):
    c = lax.axis_index("c")
    s = lax.axis_index("s")
    w = c * NS + s
    epw = idxn_hbm.shape[2] * idxn_hbm.shape[3]
    half = epw // 2
    nsub = idxn_hbm.shape[2]
    for src_i in range(2):
        idx_hbm = (idxn_hbm, idxd_hbm)[src_i]
        out_hbm = (node_out, neis_out)[src_i]
        pltpu.sync_copy(idx_hbm.at[c, s], idx_v)
        for h in range(2):
            descs = []
            for j in range(nsub // 2):
                jj = h * (nsub // 2) + j
                descs.append(pltpu.async_copy(
                    feat_hbm.at[idx_v.at[jj]],
                    rows_v.at[pl.ds(j * 128, 128)], sem))
            for d in descs:
                d.wait()
            pltpu.sync_copy(rows_v,
                            out_hbm.at[pl.ds(w * epw + h * half, half)])


def _he_gather_body(idx_hbm, h_hbm, he_out, idx_v, rows_v, sem):
    c = lax.axis_index("c")
    s = lax.axis_index("s")
    w = c * NS + s
    nsub = idx_hbm.shape[2]
    epw = nsub * 128
    pltpu.sync_copy(idx_hbm.at[c, s], idx_v)
    descs = []
    for j in range(nsub):
        descs.append(pltpu.async_copy(
            h_hbm.at[idx_v.at[j]], rows_v.at[pl.ds(j * 128, 128)], sem))
    for d in descs:
        d.wait()
    pltpu.sync_copy(rows_v, he_out.at[pl.ds(w * epw, epw)])


def _aggregate_body(out_e_hbm, loc_hbm, gidx_hbm, h_hbm, he_out,
                    idx_v, gidx_v, rows_v, grow_v, zero_v, accum, sem):
    c = lax.axis_index("c")
    s = lax.axis_index("s")
    nsub = loc_hbm.shape[1]
    eps = nsub * 128
    zslab = accum.shape[0] // NS
    for r in range(zslab):
        zero_v[r, pl.ds(0, 16)] = jnp.zeros((16,), jnp.float32)
        zero_v[r, pl.ds(16, 16)] = jnp.zeros((16,), jnp.float32)
    pltpu.sync_copy(zero_v, accum.at[pl.ds(s * zslab, zslab)])
    plsc.subcore_barrier()
    pltpu.sync_copy(out_e_hbm.at[pl.ds(s * eps, eps)], rows_v)
    pltpu.sync_copy(loc_hbm.at[s], idx_v)
    for j in range(nsub):
        pltpu.sync_copy(rows_v.at[pl.ds(j * 128, 128)],
                        accum.at[idx_v.at[j]], add=True)
    plsc.subcore_barrier()
    vhalf = 1024
    wslab = vhalf // NS
    pltpu.sync_copy(accum.at[pl.ds(c * vhalf + s * wslab, wslab)],
                    h_hbm.at[pl.ds(c * vhalf + s * wslab, wslab)])
    w = c * NS + s
    gsub = gidx_hbm.shape[2]
    epw = gsub * 128
    pltpu.sync_copy(gidx_hbm.at[c, s], gidx_v)
    descs = []
    for j in range(gsub):
        descs.append(pltpu.async_copy(
            accum.at[gidx_v.at[j]], grow_v.at[pl.ds(j * 128, 128)], sem))
    for d in descs:
        d.wait()
    pltpu.sync_copy(grow_v, he_out.at[pl.ds(w * epw, epw)])


def _sc_aggregate(out_e, loc3, gidx, V):
    E = out_e.shape[0]
    eps = E // NS
    epw = E // NW
    return pl.kernel(
        _aggregate_body,
        out_type=[jax.ShapeDtypeStruct((V, S), jnp.float32),
                  jax.ShapeDtypeStruct((E, S), jnp.float32)],
        mesh=_sc_mesh(),
        compiler_params=pltpu.CompilerParams(use_tc_tiling_on_sc=False),
        scratch_types=[pltpu.VMEM((eps // 128, 128), jnp.int32),
                       pltpu.VMEM((epw // 128, 128), jnp.int32),
                       pltpu.VMEM((eps, S), jnp.float32),
                       pltpu.VMEM((epw, S), jnp.float32),
                       pltpu.VMEM((2080 // NS, S), jnp.float32),
                       pltpu.VMEM_SHARED((2080, S), jnp.float32),
                       pltpu.SemaphoreType.DMA],
    )(out_e, loc3, gidx)


def _sc_embed_gather(idxn, idxd, feat):
    E = idxn.shape[0] * idxn.shape[1] * idxn.shape[2] * idxn.shape[3]
    epw = E // NW
    return pl.kernel(
        _embed_gather_body,
        out_type=[jax.ShapeDtypeStruct((E, FEAT), jnp.float32),
                  jax.ShapeDtypeStruct((E, FEAT), jnp.float32)],
        mesh=_sc_mesh(),
        compiler_params=pltpu.CompilerParams(use_tc_tiling_on_sc=True),
        scratch_types=[pltpu.VMEM((epw // 128, 128), jnp.int32),
                       pltpu.VMEM((epw // 2, FEAT), jnp.float32),
                       pltpu.SemaphoreType.DMA],
    )(idxn, idxd, feat)


def _sc_he_gather(idx, h):
    E = idx.shape[0] * idx.shape[1] * idx.shape[2] * idx.shape[3]
    epw = E // NW
    return pl.kernel(
        _he_gather_body,
        out_type=jax.ShapeDtypeStruct((E, S), jnp.float32),
        mesh=_sc_mesh(),
        compiler_params=pltpu.CompilerParams(use_tc_tiling_on_sc=False),
        scratch_types=[pltpu.VMEM((epw // 128, 128), jnp.int32),
                       pltpu.VMEM((epw, S), jnp.float32),
                       pltpu.SemaphoreType.DMA],
    )(idx, h)


def _agen_body(node_ref, neis_ref, et_ref, dg_ref, w1_ref, w2_ref, w3_ref,
               xib_ref, rouw_ref, roub_ref, a_ref, b_ref):
    node = node_ref[...].astype(jnp.bfloat16)
    neis = neis_ref[...].astype(jnp.bfloat16)
    acc = jnp.dot(node, w1_ref[...], preferred_element_type=jnp.float32)
    acc += jnp.dot(neis, w2_ref[...], preferred_element_type=jnp.float32)
    oh = (et_ref[...] == lax.broadcasted_iota(jnp.int32, (node.shape[0], T), 1)
          ).astype(jnp.bfloat16)
    acc += jnp.dot(oh, w3_ref[...], preferred_element_type=jnp.float32)
    acc += xib_ref[...]
    scale = (MU / S) / dg_ref[...]
    a_ref[...] = (jnp.tanh(acc) * scale).astype(jnp.bfloat16)
    b_ref[...] = jnp.tanh(
        jnp.dot(neis, rouw_ref[...], preferred_element_type=jnp.float32)
        + roub_ref[...])


def _matvec_body(a_ref, he_ref, b_ref, rep_ref, fold_ref, out_ref):
    he_rep = jnp.dot(he_ref[...].astype(jnp.bfloat16), rep_ref[...],
                     preferred_element_type=jnp.float32)
    prod = a_ref[...] * he_rep.astype(jnp.bfloat16)
    out_ref[...] = jnp.dot(prod, fold_ref[...],
                           preferred_element_type=jnp.float32) + b_ref[...]


def _final_body(h_ref, l1w_ref, l1b_ref, out_ref):
    h = h_ref[...]
    logits = jnp.sum(h * l1w_ref[...], axis=1, keepdims=True) + l1b_ref[...]
    m = jnp.max(logits)
    e = jnp.exp(logits - m)
    attn = e / jnp.sum(e)
    out_ref[...] = jnp.tanh(jnp.sum(h * attn, axis=0, keepdims=True))


def kernel(feat_Matrix, X_Node, X_Neis, edge_type_index, dg_list,
           Xi_W, Xi_b, Rou_W, Rou_b, L1_W, L1_b):
    V = feat_Matrix.shape[0]
    E = X_Node.shape[0]
    epw = E // NW
    eps = E // NS

    Xn = (X_Node - 1).astype(jnp.int32)
    Xd = (X_Neis - 1).astype(jnp.int32)
    et = (edge_type_index - 1).astype(jnp.int32)

    idxn = Xn.reshape(NC, NS, epw // 128, 128)
    idxd = Xd.reshape(NC, NS, epw // 128, 128)
    loc3 = X_Neis.astype(jnp.int32).reshape(NS, eps // 128, 128)

    node_e, neis_e = _sc_embed_gather(idxn, idxd, feat_Matrix)

    TE2 = 256
    a2, bvec = pl.pallas_call(
        _agen_body,
        grid=(E // TE2,),
        in_specs=[
            pl.BlockSpec((TE2, FEAT), lambda i: (i, 0)),
            pl.BlockSpec((TE2, FEAT), lambda i: (i, 0)),
            pl.BlockSpec((TE2, 1), lambda i: (i, 0)),
            pl.BlockSpec((TE2, 1), lambda i: (i, 0)),
            pl.BlockSpec((FEAT, S * S), lambda i: (0, 0)),
            pl.BlockSpec((FEAT, S * S), lambda i: (0, 0)),
            pl.BlockSpec((T, S * S), lambda i: (0, 0)),
            pl.BlockSpec((1, S * S), lambda i: (0, 0)),
            pl.BlockSpec((FEAT, S), lambda i: (0, 0)),
            pl.BlockSpec((1, S), lambda i: (0, 0)),
        ],
        out_specs=[pl.BlockSpec((TE2, S * S), lambda i: (i, 0)),
                   pl.BlockSpec((TE2, S), lambda i: (i, 0))],
        out_shape=[jax.ShapeDtypeStruct((E, S * S), jnp.bfloat16),
                   jax.ShapeDtypeStruct((E, S), jnp.float32)],
    )(node_e, neis_e, et[:, None], dg_list.astype(jnp.float32)[:, None],
      Xi_W[:, :FEAT].T.astype(jnp.bfloat16),
      Xi_W[:, FEAT:2 * FEAT].T.astype(jnp.bfloat16),
      Xi_W[:, 2 * FEAT:].T.astype(jnp.bfloat16),
      Xi_b[None, :], Rou_W.T.astype(jnp.bfloat16), Rou_b[None, :])
    rep_c = jnp.tile(jnp.eye(S, dtype=jnp.bfloat16), (1, S))
    fold_c = jnp.repeat(jnp.eye(S, dtype=jnp.bfloat16), S, axis=0)

    TE = 1024
    matvec = pl.pallas_call(
        _matvec_body,
        grid=(E // TE,),
        in_specs=[
            pl.BlockSpec((TE, S * S), lambda i: (i, 0)),
            pl.BlockSpec((TE, S), lambda i: (i, 0)),
            pl.BlockSpec((TE, S), lambda i: (i, 0)),
            pl.BlockSpec((S, S * S), lambda i: (0, 0)),
            pl.BlockSpec((S * S, S), lambda i: (0, 0)),
        ],
        out_specs=pl.BlockSpec((TE, S), lambda i: (i, 0)),
        out_shape=jax.ShapeDtypeStruct((E, S), jnp.float32),
    )

    h = jax.random.uniform(jax.random.key(1), (V, S), dtype=jnp.float32)
    he = _sc_he_gather(idxn, h)
    for _ in range(T):
        out_e = matvec(a2, he, bvec, rep_c, fold_c)
        h, he = _sc_aggregate(out_e, loc3, idxn, V)

    graph = pl.pallas_call(
        _final_body,
        in_specs=[pl.BlockSpec((V, S), lambda: (0, 0)),
                  pl.BlockSpec((1, S), lambda: (0, 0)),
                  pl.BlockSpec((1, 1), lambda: (0, 0))],
        out_specs=pl.BlockSpec((1, S), lambda: (0, 0)),
        out_shape=jax.ShapeDtypeStruct((1, S), jnp.float32),
    )(h, L1_W, L1_b[None, :])
    return graph.reshape(S)
